# Initial kernel scaffold; baseline (speedup 1.0000x reference)
import jax
import jax.numpy as jnp
from jax.experimental import pallas as pl


def kernel(user_embedding, item_embedding, user_neg_embedding, item_neg_embedding, eps, pos_edge_index, neg_edge_index):
    raise NotImplementedError("write your pallas kernel here")


# speedup vs baseline: 7.3783x; 7.3783x over previous
import functools

import jax
import jax.numpy as jnp
import numpy as np
from jax import lax
from jax.experimental import pallas as pl
from jax.experimental.pallas import tpu as pltpu
from jax.experimental.pallas import tpu_sc as plsc

N_USERS = 25000
N_ITEMS = 25000
HALF = 25000
PAD = 88
HALFP = HALF + PAD
NP = 2 * HALFP
D = 64
NUM_LAYERS = 2
ALPHA = np.float32(1.0 / (NUM_LAYERS + 1))
NTILES = 16
CHUNK = 128
ROWS_PER_TILE = HALFP // NTILES

E_POS = 800000
E_NEG = 400000


def _pad_chunks(e):
    blk = NTILES * CHUNK
    return ((e + blk - 1) // blk) * blk


E_POS_P = _pad_chunks(E_POS)
E_NEG_P = _pad_chunks(E_NEG)


def _remap_chunk(c, rowbuf, colbuf, rowpb, accb):
    cbase = c * HALF
    for j in range(CHUNK // 16):
        sl = pl.ds(j * 16, 16)
        col = colbuf[sl]
        row = rowbuf[sl]
        lc = col - cbase
        valid = (lc >= 0) & (lc < HALF)
        trash = HALF + (col & 63)
        accb[sl] = jnp.where(valid, lc, trash)
        rowpb[sl] = jnp.where(row >= HALF, row + PAD, row)


def _make_prop(n_edges_p):
    n_chunks = n_edges_p // (NTILES * CHUNK)
    mesh = plsc.VectorSubcoreMesh(core_axis_name="c", subcore_axis_name="s")

    @functools.partial(
        pl.kernel,
        out_type=jax.ShapeDtypeStruct((NP, D), jnp.float32),
        mesh=mesh,
        scratch_types=[
            pltpu.VMEM_SHARED((HALFP, D), jnp.float32),
            pltpu.VMEM((CHUNK,), jnp.int32),
            pltpu.VMEM((CHUNK,), jnp.int32),
            pltpu.VMEM((CHUNK,), jnp.int32),
            pltpu.VMEM((CHUNK,), jnp.int32),
            pltpu.VMEM((CHUNK, D), jnp.float32),
            pltpu.SemaphoreType.DMA,
        ],
        compiler_params=pltpu.CompilerParams(use_tc_tiling_on_sc=False),
    )
    def prop(y_hbm, rows_hbm, cols_hbm, zeros_hbm, z_hbm,
             acc, rowbuf, colbuf, rowpb, accb, rowsv, sem):
        c = lax.axis_index("c")
        s = lax.axis_index("s")
        r0 = s * ROWS_PER_TILE
        pltpu.sync_copy(zeros_hbm.at[pl.ds(r0, ROWS_PER_TILE)],
                        acc.at[pl.ds(r0, ROWS_PER_TILE)])
        plsc.subcore_barrier()

        ebase = s * (n_chunks * CHUNK)

        def body(i, carry):
            e0 = ebase + i * CHUNK
            pltpu.sync_copy(rows_hbm.at[pl.ds(e0, CHUNK)], rowbuf)
            pltpu.sync_copy(cols_hbm.at[pl.ds(e0, CHUNK)], colbuf)
            _remap_chunk(c, rowbuf, colbuf, rowpb, accb)
            pltpu.async_copy(y_hbm.at[rowpb], rowsv, sem).wait()
            pltpu.sync_copy(rowsv, acc.at[accb], add=True)
            return carry

        lax.fori_loop(0, n_chunks, body, 0)
        plsc.subcore_barrier()
        pltpu.sync_copy(acc.at[pl.ds(r0, ROWS_PER_TILE)],
                        z_hbm.at[pl.ds(c * HALFP + r0, ROWS_PER_TILE)])

    return prop


def _make_deg():
    n_chunks_p = E_POS_P // (NTILES * CHUNK)
    n_chunks_n = E_NEG_P // (NTILES * CHUNK)
    mesh = plsc.VectorSubcoreMesh(core_axis_name="c", subcore_axis_name="s")

    @functools.partial(
        pl.kernel,
        out_type=(jax.ShapeDtypeStruct((NP,), jnp.float32),
                  jax.ShapeDtypeStruct((NP,), jnp.float32)),
        mesh=mesh,
        scratch_types=[
            pltpu.VMEM_SHARED((HALFP,), jnp.float32),
            pltpu.VMEM_SHARED((HALFP,), jnp.float32),
            pltpu.VMEM((CHUNK,), jnp.int32),
            pltpu.VMEM((CHUNK,), jnp.int32),
            pltpu.VMEM((CHUNK,), jnp.int32),
            pltpu.VMEM((CHUNK,), jnp.float32),
        ],
        compiler_params=pltpu.CompilerParams(use_tc_tiling_on_sc=False),
    )
    def deg(cols_pos_hbm, cols_neg_hbm, zcol_hbm, ones_hbm,
            degp_hbm, degn_hbm,
            accp, accn, rowbuf, colbuf, accb, onesv):
        c = lax.axis_index("c")
        s = lax.axis_index("s")
        r0 = s * ROWS_PER_TILE
        pltpu.sync_copy(zcol_hbm.at[pl.ds(r0, ROWS_PER_TILE)],
                        accp.at[pl.ds(r0, ROWS_PER_TILE)])
        pltpu.sync_copy(zcol_hbm.at[pl.ds(r0, ROWS_PER_TILE)],
                        accn.at[pl.ds(r0, ROWS_PER_TILE)])
        pltpu.sync_copy(ones_hbm, onesv)
        plsc.subcore_barrier()

        def body_p(i, carry):
            e0 = s * (n_chunks_p * CHUNK) + i * CHUNK
            pltpu.sync_copy(cols_pos_hbm.at[pl.ds(e0, CHUNK)], colbuf)
            _remap_chunk(c, colbuf, colbuf, rowbuf, accb)
            pltpu.sync_copy(onesv, accp.at[accb], add=True)
            return carry

        def body_n(i, carry):
            e0 = s * (n_chunks_n * CHUNK) + i * CHUNK
            pltpu.sync_copy(cols_neg_hbm.at[pl.ds(e0, CHUNK)], colbuf)
            _remap_chunk(c, colbuf, colbuf, rowbuf, accb)
            pltpu.sync_copy(onesv, accn.at[accb], add=True)
            return carry

        lax.fori_loop(0, n_chunks_p, body_p, 0)
        lax.fori_loop(0, n_chunks_n, body_n, 0)
        plsc.subcore_barrier()
        pltpu.sync_copy(accp.at[pl.ds(r0, ROWS_PER_TILE)],
                        degp_hbm.at[pl.ds(c * HALFP + r0, ROWS_PER_TILE)])
        pltpu.sync_copy(accn.at[pl.ds(r0, ROWS_PER_TILE)],
                        degn_hbm.at[pl.ds(c * HALFP + r0, ROWS_PER_TILE)])

    return deg


TC_ROWS = NP // 16


def _vec_spec():
    return pl.BlockSpec((TC_ROWS, D), lambda i: (i, 0))


def _col_spec():
    return pl.BlockSpec((TC_ROWS, 1), lambda i: (i, 0))


def _eps_spec():
    return pl.BlockSpec(memory_space=pltpu.SMEM)


def _prep_body(degp_ref, degn_ref, x0_ref, yp_ref, yn_ref, dp_ref, dn_ref):
    degp = degp_ref[...]
    degn = degn_ref[...]
    x0 = x0_ref[...]
    dp = jnp.where(degp > 0, lax.rsqrt(degp), 0.0)
    dn = jnp.where(degn > 0, lax.rsqrt(degn), 0.0)
    dp_ref[...] = dp
    dn_ref[...] = dn
    yp_ref[...] = dp * x0
    yn_ref[...] = dn * x0


def _tc_prep(degp, degn, x0):
    out = (jax.ShapeDtypeStruct((NP, D), jnp.float32),
           jax.ShapeDtypeStruct((NP, D), jnp.float32),
           jax.ShapeDtypeStruct((NP, 1), jnp.float32),
           jax.ShapeDtypeStruct((NP, 1), jnp.float32))
    return pl.pallas_call(
        _prep_body,
        grid=(16,),
        in_specs=[_col_spec(), _col_spec(), _vec_spec()],
        out_specs=(_vec_spec(), _vec_spec(), _col_spec(), _col_spec()),
        out_shape=out,
    )(degp, degn, x0)


def _mid_body(eps_ref, zp_ref, zn_ref, yp_ref, yn_ref, dp_ref, dn_ref,
              x0_ref, egn_ref, y1p_ref, y1n_ref, pp_ref, np_ref):
    e0 = 1.0 + eps_ref[0]
    dp = dp_ref[...]
    dn = dn_ref[...]
    op0 = dp * (zp_ref[...] + e0 * yp_ref[...])
    on0 = dn * (zn_ref[...] + e0 * yn_ref[...])
    y1p_ref[...] = dp * op0
    y1n_ref[...] = dp * on0
    pp_ref[...] = ALPHA * (x0_ref[...] + op0)
    np_ref[...] = ALPHA * (egn_ref[...] + on0)


def _tc_mid(eps, zp0, zn0, yp0, yn0, dp, dn, x0, egn):
    out = tuple(jax.ShapeDtypeStruct((NP, D), jnp.float32) for _ in range(4))
    return pl.pallas_call(
        _mid_body,
        grid=(16,),
        in_specs=[_eps_spec(), _vec_spec(), _vec_spec(), _vec_spec(),
                  _vec_spec(), _col_spec(), _col_spec(), _vec_spec(),
                  _vec_spec()],
        out_specs=(_vec_spec(), _vec_spec(), _vec_spec(), _vec_spec()),
        out_shape=out,
    )(eps, zp0, zn0, yp0, yn0, dp, dn, x0, egn)


def _fin_body(eps_ref, zp1_ref, zn1_ref, y1p_ref, y1n_ref, dp_ref,
              pp_ref, np_ref, po_ref, no_ref):
    e1 = 1.0 + eps_ref[1]
    dp = dp_ref[...]
    op1 = dp * (zp1_ref[...] + e1 * y1p_ref[...])
    on1 = dp * (zn1_ref[...] + e1 * y1n_ref[...])
    po_ref[...] = pp_ref[...] + ALPHA * op1
    no_ref[...] = np_ref[...] + ALPHA * on1


def _tc_fin(eps, zp1, zn1, y1p, y1n, dp, pp, npart):
    out = (jax.ShapeDtypeStruct((NP, D), jnp.float32),
           jax.ShapeDtypeStruct((NP, D), jnp.float32))
    return pl.pallas_call(
        _fin_body,
        grid=(16,),
        in_specs=[_eps_spec(), _vec_spec(), _vec_spec(), _vec_spec(),
                  _vec_spec(), _col_spec(), _vec_spec(), _vec_spec()],
        out_specs=(_vec_spec(), _vec_spec()),
        out_shape=out,
    )(eps, zp1, zn1, y1p, y1n, dp, pp, npart)


_prop_pos = _make_prop(E_POS_P)
_prop_neg = _make_prop(E_NEG_P)
_deg_kernel = _make_deg()


def _pad_nodes(a, b):
    z = jnp.zeros((PAD, D), jnp.float32)
    return jnp.concatenate([a, z, b, z], axis=0)


def _pad_edges(edge_index, n_pad):
    e = edge_index.shape[1]
    extra = n_pad - e
    rows = jnp.concatenate(
        [edge_index[0], (jnp.arange(extra, dtype=jnp.int32) % 16)])
    cols = jnp.concatenate(
        [edge_index[1], jnp.full((extra,), -1, jnp.int32)])
    return rows, cols


def kernel(user_embedding, item_embedding, user_neg_embedding,
           item_neg_embedding, eps, pos_edge_index, neg_edge_index):
    x0 = _pad_nodes(user_embedding, item_embedding)
    egn = _pad_nodes(user_neg_embedding, item_neg_embedding)
    rows_p, cols_p = _pad_edges(pos_edge_index, E_POS_P)
    rows_n, cols_n = _pad_edges(neg_edge_index, E_NEG_P)

    zeros64 = jnp.zeros((HALFP, D), jnp.float32)
    zcol = jnp.zeros((HALFP,), jnp.float32)
    ones128 = jnp.ones((CHUNK,), jnp.float32)

    degp, degn = _deg_kernel(cols_p, cols_n, zcol, ones128)
    yp0, yn0, dp, dn = _tc_prep(degp.reshape(NP, 1), degn.reshape(NP, 1), x0)

    zp0 = _prop_pos(yp0, rows_p, cols_p, zeros64)
    zn0 = _prop_neg(yn0, rows_n, cols_n, zeros64)

    y1p, y1n, pp, npart = _tc_mid(eps, zp0, zn0, yp0, yn0, dp, dn, x0, egn)

    zp1 = _prop_pos(y1p, rows_p, cols_p, zeros64)
    zn1 = _prop_pos(y1n, rows_p, cols_p, zeros64)

    po, no = _tc_fin(eps, zp1, zn1, y1p, y1n, dp, pp, npart)

    pos_emb = jnp.concatenate([po[:HALF], po[HALFP:HALFP + HALF]], axis=0)
    neg_emb = jnp.concatenate([no[:HALF], no[HALFP:HALFP + HALF]], axis=0)
    return pos_emb, neg_emb


# speedup vs baseline: 14.9890x; 2.0315x over previous
import functools

import jax
import jax.numpy as jnp
import numpy as np
from jax import lax
from jax.experimental import pallas as pl
from jax.experimental.pallas import tpu as pltpu
from jax.experimental.pallas import tpu_sc as plsc

N_USERS = 25000
N_ITEMS = 25000
HALF = 25000
PAD = 88
HALFP = HALF + PAD
NP = 2 * HALFP
D = 64
NUM_LAYERS = 2
ALPHA = np.float32(1.0 / (NUM_LAYERS + 1))
NTILES = 16
CHUNK = 128
ROWS_PER_TILE = HALFP // NTILES

E_POS = 800000
E_NEG = 400000


def _pad_chunks(e):
    blk = NTILES * CHUNK * 2
    return ((e + blk - 1) // blk) * blk


E_SLACK = 4 * CHUNK
E_POS_P = _pad_chunks(E_POS)
E_NEG_P = _pad_chunks(E_NEG)


def _remap_chunk(c, rowbuf, colbuf, rowpb, accb):
    cbase = c * HALF
    for j in range(CHUNK // 16):
        sl = pl.ds(j * 16, 16)
        col = colbuf[sl]
        row = rowbuf[sl]
        lc = col - cbase
        valid = (lc >= 0) & (lc < HALF)
        trash = HALF + (col & 63)
        accb[sl] = jnp.where(valid, lc, trash)
        rowpb[sl] = jnp.where(row >= HALF, row + PAD, row)


def _make_prop2(n_edges_a, n_edges_b):
    mesh = plsc.VectorSubcoreMesh(core_axis_name="c", subcore_axis_name="s")

    @functools.partial(
        pl.kernel,
        out_type=(jax.ShapeDtypeStruct((NP, D), jnp.float32),
                  jax.ShapeDtypeStruct((NP, D), jnp.float32)),
        mesh=mesh,
        scratch_types=[
            pltpu.VMEM_SHARED((HALFP, D), jnp.float32),
            pltpu.VMEM((2, CHUNK), jnp.int32),
            pltpu.VMEM((2, CHUNK), jnp.int32),
            pltpu.VMEM((CHUNK,), jnp.int32),
            pltpu.VMEM((CHUNK,), jnp.int32),
            pltpu.VMEM((CHUNK,), jnp.int32),
            pltpu.VMEM((CHUNK,), jnp.int32),
            pltpu.VMEM((CHUNK, D), jnp.float32),
            pltpu.VMEM((CHUNK, D), jnp.float32),
            pltpu.SemaphoreType.DMA,
            pltpu.SemaphoreType.DMA,
            pltpu.SemaphoreType.DMA,
            pltpu.SemaphoreType.DMA,
        ],
        compiler_params=pltpu.CompilerParams(use_tc_tiling_on_sc=False),
    )
    def prop(ya_hbm, edges_a_hbm, yb_hbm, edges_b_hbm, zeros_hbm,
             za_hbm, zb_hbm,
             acc, ebuf0, ebuf1, rowpb0, accb0, rowpb1, accb1,
             rowsv0, rowsv1, esem0, esem1, gsem0, gsem1):
        c = lax.axis_index("c")
        s = lax.axis_index("s")
        r0 = s * ROWS_PER_TILE
        cbase = c * HALF

        def remap(ebuf, rowpb, accb):
            for j in range(CHUNK // 16):
                sl = pl.ds(j * 16, 16)
                col = ebuf[1, sl]
                row = ebuf[0, sl]
                lc = col - cbase
                valid = (lc >= 0) & (lc < HALF)
                trash = HALF + (col & 63)
                accb[sl] = jnp.where(valid, lc, trash)
                rowpb[sl] = jnp.where(row >= HALF, row + PAD, row)

        def one_pass(y_hbm, edges_hbm, z_hbm, n_chunks):
            ebase = s * (n_chunks * CHUNK)

            def idx_copy(chunk, ebuf, esem):
                return pltpu.async_copy(
                    edges_hbm.at[:, pl.ds(ebase + chunk * CHUNK, CHUNK)],
                    ebuf, esem)

            def gather(rowpb, rowsv, gsem):
                return pltpu.async_copy(y_hbm.at[rowpb], rowsv, gsem)

            init = pltpu.async_copy(zeros_hbm.at[pl.ds(r0, ROWS_PER_TILE)],
                                    acc.at[pl.ds(r0, ROWS_PER_TILE)], gsem0)
            i0 = idx_copy(0, ebuf0, esem0)
            idx_copy(1, ebuf1, esem1)
            init.wait()
            plsc.subcore_barrier()
            i0.wait()
            remap(ebuf0, rowpb0, accb0)
            idx_copy(2, ebuf0, esem0)
            gather(rowpb0, rowsv0, gsem0)

            def body(g, carry):
                c1 = 2 * g + 1
                pltpu.make_async_copy(edges_hbm.at[:, pl.ds(0, CHUNK)],
                                      ebuf1, esem1).wait()
                remap(ebuf1, rowpb1, accb1)
                idx_copy(c1 + 2, ebuf1, esem1)
                gather(rowpb1, rowsv1, gsem1)
                pltpu.make_async_copy(y_hbm.at[rowpb0], rowsv0, gsem0).wait()
                pltpu.sync_copy(rowsv0, acc.at[accb0], add=True)
                pltpu.make_async_copy(edges_hbm.at[:, pl.ds(0, CHUNK)],
                                      ebuf0, esem0).wait()
                remap(ebuf0, rowpb0, accb0)
                idx_copy(c1 + 3, ebuf0, esem0)
                gather(rowpb0, rowsv0, gsem0)
                pltpu.make_async_copy(y_hbm.at[rowpb1], rowsv1, gsem1).wait()
                pltpu.sync_copy(rowsv1, acc.at[accb1], add=True)
                return carry

            lax.fori_loop(0, n_chunks // 2, body, 0)
            pltpu.make_async_copy(y_hbm.at[rowpb0], rowsv0, gsem0).wait()
            pltpu.make_async_copy(edges_hbm.at[:, pl.ds(0, CHUNK)],
                                  ebuf0, esem0).wait()
            pltpu.make_async_copy(edges_hbm.at[:, pl.ds(0, CHUNK)],
                                  ebuf1, esem1).wait()
            plsc.subcore_barrier()
            pltpu.sync_copy(acc.at[pl.ds(r0, ROWS_PER_TILE)],
                            z_hbm.at[pl.ds(c * HALFP + r0, ROWS_PER_TILE)])

        one_pass(ya_hbm, edges_a_hbm, za_hbm, n_edges_a // (NTILES * CHUNK))
        plsc.subcore_barrier()
        one_pass(yb_hbm, edges_b_hbm, zb_hbm, n_edges_b // (NTILES * CHUNK))

    return prop


def _make_deg():
    n_chunks_p = E_POS_P // (NTILES * CHUNK)
    n_chunks_n = E_NEG_P // (NTILES * CHUNK)
    mesh = plsc.VectorSubcoreMesh(core_axis_name="c", subcore_axis_name="s")

    @functools.partial(
        pl.kernel,
        out_type=(jax.ShapeDtypeStruct((NP,), jnp.float32),
                  jax.ShapeDtypeStruct((NP,), jnp.float32)),
        mesh=mesh,
        scratch_types=[
            pltpu.VMEM_SHARED((HALFP,), jnp.float32),
            pltpu.VMEM_SHARED((HALFP,), jnp.float32),
            pltpu.VMEM((CHUNK,), jnp.int32),
            pltpu.VMEM((CHUNK,), jnp.int32),
            pltpu.VMEM((CHUNK,), jnp.int32),
            pltpu.VMEM((CHUNK,), jnp.float32),
        ],
        compiler_params=pltpu.CompilerParams(use_tc_tiling_on_sc=False),
    )
    def deg(cols_pos_hbm, cols_neg_hbm, zcol_hbm, ones_hbm,
            degp_hbm, degn_hbm,
            accp, accn, rowbuf, colbuf, accb, onesv):
        c = lax.axis_index("c")
        s = lax.axis_index("s")
        r0 = s * ROWS_PER_TILE
        pltpu.sync_copy(zcol_hbm.at[pl.ds(r0, ROWS_PER_TILE)],
                        accp.at[pl.ds(r0, ROWS_PER_TILE)])
        pltpu.sync_copy(zcol_hbm.at[pl.ds(r0, ROWS_PER_TILE)],
                        accn.at[pl.ds(r0, ROWS_PER_TILE)])
        pltpu.sync_copy(ones_hbm, onesv)
        plsc.subcore_barrier()

        def body_p(i, carry):
            e0 = s * (n_chunks_p * CHUNK) + i * CHUNK
            pltpu.sync_copy(cols_pos_hbm.at[1, pl.ds(e0, CHUNK)], colbuf)
            _remap_chunk(c, colbuf, colbuf, rowbuf, accb)
            pltpu.sync_copy(onesv, accp.at[accb], add=True)
            return carry

        def body_n(i, carry):
            e0 = s * (n_chunks_n * CHUNK) + i * CHUNK
            pltpu.sync_copy(cols_neg_hbm.at[1, pl.ds(e0, CHUNK)], colbuf)
            _remap_chunk(c, colbuf, colbuf, rowbuf, accb)
            pltpu.sync_copy(onesv, accn.at[accb], add=True)
            return carry

        lax.fori_loop(0, n_chunks_p, body_p, 0)
        lax.fori_loop(0, n_chunks_n, body_n, 0)
        plsc.subcore_barrier()
        pltpu.sync_copy(accp.at[pl.ds(r0, ROWS_PER_TILE)],
                        degp_hbm.at[pl.ds(c * HALFP + r0, ROWS_PER_TILE)])
        pltpu.sync_copy(accn.at[pl.ds(r0, ROWS_PER_TILE)],
                        degn_hbm.at[pl.ds(c * HALFP + r0, ROWS_PER_TILE)])

    return deg


TC_ROWS = NP // 32


def _vec_spec():
    return pl.BlockSpec((TC_ROWS, D), lambda i: (i, 0))


def _col_spec():
    return pl.BlockSpec((TC_ROWS, 1), lambda i: (i, 0))


def _eps_spec():
    return pl.BlockSpec(memory_space=pltpu.SMEM)


def _prep_body(degp_ref, degn_ref, x0_ref, yp_ref, yn_ref, dp_ref, dn_ref):
    degp = degp_ref[...]
    degn = degn_ref[...]
    x0 = x0_ref[...]
    dp = jnp.where(degp > 0, lax.rsqrt(degp), 0.0)
    dn = jnp.where(degn > 0, lax.rsqrt(degn), 0.0)
    dp_ref[...] = dp
    dn_ref[...] = dn
    yp_ref[...] = dp * x0
    yn_ref[...] = dn * x0


def _tc_prep(degp, degn, x0):
    out = (jax.ShapeDtypeStruct((NP, D), jnp.float32),
           jax.ShapeDtypeStruct((NP, D), jnp.float32),
           jax.ShapeDtypeStruct((NP, 1), jnp.float32),
           jax.ShapeDtypeStruct((NP, 1), jnp.float32))
    return pl.pallas_call(
        _prep_body,
        grid=(32,),
        in_specs=[_col_spec(), _col_spec(), _vec_spec()],
        out_specs=(_vec_spec(), _vec_spec(), _col_spec(), _col_spec()),
        out_shape=out,
    )(degp, degn, x0)


def _mid_body(eps_ref, zp_ref, zn_ref, yp_ref, yn_ref, dp_ref, dn_ref,
              x0_ref, egn_ref, y1p_ref, y1n_ref, pp_ref, np_ref):
    e0 = 1.0 + eps_ref[0]
    dp = dp_ref[...]
    dn = dn_ref[...]
    op0 = dp * (zp_ref[...] + e0 * yp_ref[...])
    on0 = dn * (zn_ref[...] + e0 * yn_ref[...])
    y1p_ref[...] = dp * op0
    y1n_ref[...] = dp * on0
    pp_ref[...] = ALPHA * (x0_ref[...] + op0)
    np_ref[...] = ALPHA * (egn_ref[...] + on0)


def _tc_mid(eps, zp0, zn0, yp0, yn0, dp, dn, x0, egn):
    out = tuple(jax.ShapeDtypeStruct((NP, D), jnp.float32) for _ in range(4))
    return pl.pallas_call(
        _mid_body,
        grid=(32,),
        in_specs=[_eps_spec(), _vec_spec(), _vec_spec(), _vec_spec(),
                  _vec_spec(), _col_spec(), _col_spec(), _vec_spec(),
                  _vec_spec()],
        out_specs=(_vec_spec(), _vec_spec(), _vec_spec(), _vec_spec()),
        out_shape=out,
    )(eps, zp0, zn0, yp0, yn0, dp, dn, x0, egn)


def _fin_body(eps_ref, zp1_ref, zn1_ref, y1p_ref, y1n_ref, dp_ref,
              pp_ref, np_ref, po_ref, no_ref):
    e1 = 1.0 + eps_ref[1]
    dp = dp_ref[...]
    op1 = dp * (zp1_ref[...] + e1 * y1p_ref[...])
    on1 = dp * (zn1_ref[...] + e1 * y1n_ref[...])
    po_ref[...] = pp_ref[...] + ALPHA * op1
    no_ref[...] = np_ref[...] + ALPHA * on1


def _tc_fin(eps, zp1, zn1, y1p, y1n, dp, pp, npart):
    out = (jax.ShapeDtypeStruct((NP, D), jnp.float32),
           jax.ShapeDtypeStruct((NP, D), jnp.float32))
    return pl.pallas_call(
        _fin_body,
        grid=(32,),
        in_specs=[_eps_spec(), _vec_spec(), _vec_spec(), _vec_spec(),
                  _vec_spec(), _col_spec(), _vec_spec(), _vec_spec()],
        out_specs=(_vec_spec(), _vec_spec()),
        out_shape=out,
    )(eps, zp1, zn1, y1p, y1n, dp, pp, npart)


_prop_l0 = _make_prop2(E_POS_P, E_NEG_P)
_prop_l1 = _make_prop2(E_POS_P, E_POS_P)
_deg_kernel = _make_deg()


def _pad_nodes(a, b):
    z = jnp.zeros((PAD, D), jnp.float32)
    return jnp.concatenate([a, z, b, z], axis=0)


def _pad_edges(edge_index, n_pad):
    e = edge_index.shape[1]
    extra = n_pad + E_SLACK - e
    rows = (jnp.arange(extra, dtype=jnp.int32) % 16)[None]
    cols = jnp.full((1, extra), -1, jnp.int32)
    return jnp.concatenate(
        [edge_index, jnp.concatenate([rows, cols], axis=0)], axis=1)


def kernel(user_embedding, item_embedding, user_neg_embedding,
           item_neg_embedding, eps, pos_edge_index, neg_edge_index):
    x0 = _pad_nodes(user_embedding, item_embedding)
    egn = _pad_nodes(user_neg_embedding, item_neg_embedding)
    edges_p = _pad_edges(pos_edge_index, E_POS_P)
    edges_n = _pad_edges(neg_edge_index, E_NEG_P)

    zeros64 = jnp.zeros((HALFP, D), jnp.float32)
    zcol = jnp.zeros((HALFP,), jnp.float32)
    ones128 = jnp.ones((CHUNK,), jnp.float32)

    degp, degn = _deg_kernel(edges_p, edges_n, zcol, ones128)
    yp0, yn0, dp, dn = _tc_prep(degp.reshape(NP, 1), degn.reshape(NP, 1), x0)

    zp0, zn0 = _prop_l0(yp0, edges_p, yn0, edges_n, zeros64)

    y1p, y1n, pp, npart = _tc_mid(eps, zp0, zn0, yp0, yn0, dp, dn, x0, egn)

    zp1, zn1 = _prop_l1(y1p, edges_p, y1n, edges_p, zeros64)

    po, no = _tc_fin(eps, zp1, zn1, y1p, y1n, dp, pp, npart)

    pos_emb = jnp.concatenate([po[:HALF], po[HALFP:HALFP + HALF]], axis=0)
    neg_emb = jnp.concatenate([no[:HALF], no[HALFP:HALFP + HALF]], axis=0)
    return pos_emb, neg_emb


# speedup vs baseline: 18.5805x; 1.2396x over previous
import functools

import jax
import jax.numpy as jnp
import numpy as np
from jax import lax
from jax.experimental import pallas as pl
from jax.experimental.pallas import tpu as pltpu
from jax.experimental.pallas import tpu_sc as plsc

N_USERS = 25000
N_ITEMS = 25000
HALF = 25000
PAD = 88
HALFP = HALF + PAD
NP = 2 * HALFP
D = 64
NUM_LAYERS = 2
ALPHA = np.float32(1.0 / (NUM_LAYERS + 1))
NTILES = 16
CHUNK = 128
ROWS_PER_TILE = HALFP // NTILES

E_POS = 800000
E_NEG = 400000


NSLOT = 3


def _pad_chunks(e):
    blk = NTILES * CHUNK * NSLOT
    return ((e + blk - 1) // blk) * blk


E_SLACK = NSLOT * CHUNK
E_POS_P = _pad_chunks(E_POS)
E_NEG_P = _pad_chunks(E_NEG)


def _make_prop2(n_edges_a, n_edges_b):
    mesh = plsc.VectorSubcoreMesh(core_axis_name="c", subcore_axis_name="s")

    @functools.partial(
        pl.kernel,
        out_type=(jax.ShapeDtypeStruct((NP, D), jnp.float32),
                  jax.ShapeDtypeStruct((NP, D), jnp.float32)),
        mesh=mesh,
        scratch_types=(
            [pltpu.VMEM_SHARED((HALFP, D), jnp.float32)]
            + [pltpu.VMEM((2, CHUNK), jnp.int32) for _ in range(NSLOT)]
            + [pltpu.VMEM((CHUNK,), jnp.int32) for _ in range(2 * NSLOT)]
            + [pltpu.VMEM((CHUNK, D), jnp.float32) for _ in range(NSLOT)]
            + [pltpu.SemaphoreType.DMA for _ in range(3 * NSLOT)]
        ),
        compiler_params=pltpu.CompilerParams(use_tc_tiling_on_sc=False),
    )
    def prop(ya_hbm, edges_a_hbm, yb_hbm, edges_b_hbm, zeros_hbm,
             za_hbm, zb_hbm, acc, *slots):
        c = lax.axis_index("c")
        s = lax.axis_index("s")
        r0 = s * ROWS_PER_TILE
        cbase = c * HALF
        ebufs = slots[0:NSLOT]
        rowpbs = slots[NSLOT:2 * NSLOT]
        accbs = slots[2 * NSLOT:3 * NSLOT]
        rowsvs = slots[3 * NSLOT:4 * NSLOT]
        esems = slots[4 * NSLOT:5 * NSLOT]
        gsems = slots[5 * NSLOT:6 * NSLOT]
        ssems = slots[6 * NSLOT:7 * NSLOT]

        def remap(ebuf, rowpb, accb):
            for j in range(CHUNK // 16):
                sl = pl.ds(j * 16, 16)
                col = ebuf[1, sl]
                row = ebuf[0, sl]
                lc = col - cbase
                valid = (lc >= 0) & (lc < HALF)
                trash = HALF + (col & 63)
                accb[sl] = jnp.where(valid, lc, trash)
                rowpb[sl] = jnp.where(row >= HALF, row + PAD, row)

        def one_pass(y_hbm, edges_hbm, z_hbm, n_chunks):
            ebase = s * (n_chunks * CHUNK)

            def idx_copy(chunk, b):
                pltpu.async_copy(
                    edges_hbm.at[:, pl.ds(ebase + chunk * CHUNK, CHUNK)],
                    ebufs[b], esems[b])

            def idx_wait(b):
                pltpu.make_async_copy(edges_hbm.at[:, pl.ds(0, CHUNK)],
                                      ebufs[b], esems[b]).wait()

            def gather(b):
                pltpu.async_copy(y_hbm.at[rowpbs[b]], rowsvs[b], gsems[b])

            def gather_wait(b):
                pltpu.make_async_copy(y_hbm.at[rowpbs[b]], rowsvs[b],
                                      gsems[b]).wait()

            def scatter(b):
                pltpu.async_copy(rowsvs[b], acc.at[accbs[b]], ssems[b],
                                 add=True)

            def scatter_wait(b):
                pltpu.make_async_copy(rowsvs[b], acc.at[accbs[b]],
                                      ssems[b]).wait()

            init = pltpu.async_copy(zeros_hbm.at[pl.ds(r0, ROWS_PER_TILE)],
                                    acc.at[pl.ds(r0, ROWS_PER_TILE)],
                                    gsems[0])
            for b in range(NSLOT):
                idx_copy(b, b)
            init.wait()
            plsc.subcore_barrier()
            for b in range(NSLOT):
                idx_wait(b)
                remap(ebufs[b], rowpbs[b], accbs[b])
                idx_copy(b + NSLOT, b)
                gather(b)
                if b > 0:
                    gather_wait(b - 1)
                    scatter(b - 1)

            def body(g, carry):
                i0 = NSLOT * g
                for b in range(NSLOT):
                    i = i0 + b
                    idx_wait(b)
                    scatter_wait(b)
                    remap(ebufs[b], rowpbs[b], accbs[b])
                    idx_copy(i + NSLOT, b)
                    gather(b)
                    pb = (b - 1) % NSLOT
                    gather_wait(pb)
                    scatter(pb)
                return carry

            lax.fori_loop(1, n_chunks // NSLOT, body, 0)
            gather_wait(NSLOT - 1)
            scatter(NSLOT - 1)
            for b in range(NSLOT):
                scatter_wait(b)
                idx_wait(b)
            plsc.subcore_barrier()
            pltpu.sync_copy(acc.at[pl.ds(r0, ROWS_PER_TILE)],
                            z_hbm.at[pl.ds(c * HALFP + r0, ROWS_PER_TILE)])

        one_pass(ya_hbm, edges_a_hbm, za_hbm, n_edges_a // (NTILES * CHUNK))
        plsc.subcore_barrier()
        one_pass(yb_hbm, edges_b_hbm, zb_hbm, n_edges_b // (NTILES * CHUNK))

    return prop


def _make_deg():
    n_chunks_p = E_POS_P // (NTILES * CHUNK)
    n_chunks_n = E_NEG_P // (NTILES * CHUNK)
    mesh = plsc.VectorSubcoreMesh(core_axis_name="c", subcore_axis_name="s")

    @functools.partial(
        pl.kernel,
        out_type=(jax.ShapeDtypeStruct((NP,), jnp.float32),
                  jax.ShapeDtypeStruct((NP,), jnp.float32)),
        mesh=mesh,
        scratch_types=(
            [pltpu.VMEM_SHARED((HALFP,), jnp.float32) for _ in range(2)]
            + [pltpu.VMEM((CHUNK,), jnp.int32) for _ in range(2 * NSLOT)]
            + [pltpu.VMEM((CHUNK,), jnp.float32)]
            + [pltpu.SemaphoreType.DMA for _ in range(2 * NSLOT)]
        ),
        compiler_params=pltpu.CompilerParams(use_tc_tiling_on_sc=False),
    )
    def deg(cols_pos_hbm, cols_neg_hbm, zcol_hbm, ones_hbm,
            degp_hbm, degn_hbm,
            accp, accn, *rest):
        c = lax.axis_index("c")
        s = lax.axis_index("s")
        r0 = s * ROWS_PER_TILE
        cbase = c * HALF
        cbufs = rest[0:NSLOT]
        accbs = rest[NSLOT:2 * NSLOT]
        onesv = rest[2 * NSLOT]
        esems = rest[2 * NSLOT + 1:3 * NSLOT + 1]
        ssems = rest[3 * NSLOT + 1:4 * NSLOT + 1]
        pltpu.sync_copy(zcol_hbm.at[pl.ds(r0, ROWS_PER_TILE)],
                        accp.at[pl.ds(r0, ROWS_PER_TILE)])
        pltpu.sync_copy(zcol_hbm.at[pl.ds(r0, ROWS_PER_TILE)],
                        accn.at[pl.ds(r0, ROWS_PER_TILE)])
        pltpu.sync_copy(ones_hbm, onesv)

        def cremap(cbuf, accb):
            for j in range(CHUNK // 16):
                sl = pl.ds(j * 16, 16)
                col = cbuf[sl]
                lc = col - cbase
                valid = (lc >= 0) & (lc < HALF)
                accb[sl] = jnp.where(valid, lc, HALF + (col & 63))

        def one_pass(cols_hbm, acc, n_chunks):
            ebase = s * (n_chunks * CHUNK)

            def idx_copy(chunk, b):
                pltpu.async_copy(
                    cols_hbm.at[1, pl.ds(ebase + chunk * CHUNK, CHUNK)],
                    cbufs[b], esems[b])

            def idx_wait(b):
                pltpu.make_async_copy(cols_hbm.at[1, pl.ds(0, CHUNK)],
                                      cbufs[b], esems[b]).wait()

            def scatter(b):
                pltpu.async_copy(onesv, acc.at[accbs[b]], ssems[b], add=True)

            def scatter_wait(b):
                pltpu.make_async_copy(onesv, acc.at[accbs[b]],
                                      ssems[b]).wait()

            for b in range(NSLOT):
                idx_copy(b, b)
            for b in range(NSLOT):
                idx_wait(b)
                cremap(cbufs[b], accbs[b])
                idx_copy(b + NSLOT, b)
                scatter(b)

            def body(g, carry):
                for b in range(NSLOT):
                    i = NSLOT * g + b
                    idx_wait(b)
                    scatter_wait(b)
                    cremap(cbufs[b], accbs[b])
                    idx_copy(i + NSLOT, b)
                    scatter(b)
                return carry

            lax.fori_loop(1, n_chunks // NSLOT, body, 0)
            for b in range(NSLOT):
                scatter_wait(b)
                idx_wait(b)

        plsc.subcore_barrier()
        one_pass(cols_pos_hbm, accp, n_chunks_p)
        one_pass(cols_neg_hbm, accn, n_chunks_n)
        plsc.subcore_barrier()
        pltpu.sync_copy(accp.at[pl.ds(r0, ROWS_PER_TILE)],
                        degp_hbm.at[pl.ds(c * HALFP + r0, ROWS_PER_TILE)])
        pltpu.sync_copy(accn.at[pl.ds(r0, ROWS_PER_TILE)],
                        degn_hbm.at[pl.ds(c * HALFP + r0, ROWS_PER_TILE)])

    return deg


TC_ROWS = NP // 32


def _vec_spec():
    return pl.BlockSpec((TC_ROWS, D), lambda i: (i, 0))


def _col_spec():
    return pl.BlockSpec((TC_ROWS, 1), lambda i: (i, 0))


def _eps_spec():
    return pl.BlockSpec(memory_space=pltpu.SMEM)


def _prep_body(degp_ref, degn_ref, x0_ref, yp_ref, yn_ref, dp_ref, dn_ref):
    degp = degp_ref[...]
    degn = degn_ref[...]
    x0 = x0_ref[...]
    dp = jnp.where(degp > 0, lax.rsqrt(degp), 0.0)
    dn = jnp.where(degn > 0, lax.rsqrt(degn), 0.0)
    dp_ref[...] = dp
    dn_ref[...] = dn
    yp_ref[...] = dp * x0
    yn_ref[...] = dn * x0


def _tc_prep(degp, degn, x0):
    out = (jax.ShapeDtypeStruct((NP, D), jnp.float32),
           jax.ShapeDtypeStruct((NP, D), jnp.float32),
           jax.ShapeDtypeStruct((NP, 1), jnp.float32),
           jax.ShapeDtypeStruct((NP, 1), jnp.float32))
    return pl.pallas_call(
        _prep_body,
        grid=(32,),
        in_specs=[_col_spec(), _col_spec(), _vec_spec()],
        out_specs=(_vec_spec(), _vec_spec(), _col_spec(), _col_spec()),
        out_shape=out,
    )(degp, degn, x0)


def _mid_body(eps_ref, zp_ref, zn_ref, yp_ref, yn_ref, dp_ref, dn_ref,
              x0_ref, egn_ref, y1p_ref, y1n_ref, pp_ref, np_ref):
    e0 = 1.0 + eps_ref[0]
    dp = dp_ref[...]
    dn = dn_ref[...]
    op0 = dp * (zp_ref[...] + e0 * yp_ref[...])
    on0 = dn * (zn_ref[...] + e0 * yn_ref[...])
    y1p_ref[...] = dp * op0
    y1n_ref[...] = dp * on0
    pp_ref[...] = ALPHA * (x0_ref[...] + op0)
    np_ref[...] = ALPHA * (egn_ref[...] + on0)


def _tc_mid(eps, zp0, zn0, yp0, yn0, dp, dn, x0, egn):
    out = tuple(jax.ShapeDtypeStruct((NP, D), jnp.float32) for _ in range(4))
    return pl.pallas_call(
        _mid_body,
        grid=(32,),
        in_specs=[_eps_spec(), _vec_spec(), _vec_spec(), _vec_spec(),
                  _vec_spec(), _col_spec(), _col_spec(), _vec_spec(),
                  _vec_spec()],
        out_specs=(_vec_spec(), _vec_spec(), _vec_spec(), _vec_spec()),
        out_shape=out,
    )(eps, zp0, zn0, yp0, yn0, dp, dn, x0, egn)


def _fin_body(eps_ref, zp1_ref, zn1_ref, y1p_ref, y1n_ref, dp_ref,
              pp_ref, np_ref, po_ref, no_ref):
    e1 = 1.0 + eps_ref[1]
    dp = dp_ref[...]
    op1 = dp * (zp1_ref[...] + e1 * y1p_ref[...])
    on1 = dp * (zn1_ref[...] + e1 * y1n_ref[...])
    po_ref[...] = pp_ref[...] + ALPHA * op1
    no_ref[...] = np_ref[...] + ALPHA * on1


def _tc_fin(eps, zp1, zn1, y1p, y1n, dp, pp, npart):
    out = (jax.ShapeDtypeStruct((NP, D), jnp.float32),
           jax.ShapeDtypeStruct((NP, D), jnp.float32))
    return pl.pallas_call(
        _fin_body,
        grid=(32,),
        in_specs=[_eps_spec(), _vec_spec(), _vec_spec(), _vec_spec(),
                  _vec_spec(), _col_spec(), _vec_spec(), _vec_spec()],
        out_specs=(_vec_spec(), _vec_spec()),
        out_shape=out,
    )(eps, zp1, zn1, y1p, y1n, dp, pp, npart)


_prop_l0 = _make_prop2(E_POS_P, E_NEG_P)
_prop_l1 = _make_prop2(E_POS_P, E_POS_P)
_deg_kernel = _make_deg()


def _pad_nodes(a, b):
    z = jnp.zeros((PAD, D), jnp.float32)
    return jnp.concatenate([a, z, b, z], axis=0)


def _pad_edges(edge_index, n_pad):
    e = edge_index.shape[1]
    extra = n_pad + E_SLACK - e
    rows = (jnp.arange(extra, dtype=jnp.int32) % 16)[None]
    cols = jnp.full((1, extra), -1, jnp.int32)
    return jnp.concatenate(
        [edge_index, jnp.concatenate([rows, cols], axis=0)], axis=1)


def kernel(user_embedding, item_embedding, user_neg_embedding,
           item_neg_embedding, eps, pos_edge_index, neg_edge_index):
    x0 = _pad_nodes(user_embedding, item_embedding)
    egn = _pad_nodes(user_neg_embedding, item_neg_embedding)
    edges_p = _pad_edges(pos_edge_index, E_POS_P)
    edges_n = _pad_edges(neg_edge_index, E_NEG_P)

    zeros64 = jnp.zeros((HALFP, D), jnp.float32)
    zcol = jnp.zeros((HALFP,), jnp.float32)
    ones128 = jnp.ones((CHUNK,), jnp.float32)

    degp, degn = _deg_kernel(edges_p, edges_n, zcol, ones128)
    yp0, yn0, dp, dn = _tc_prep(degp.reshape(NP, 1), degn.reshape(NP, 1), x0)

    zp0, zn0 = _prop_l0(yp0, edges_p, yn0, edges_n, zeros64)

    y1p, y1n, pp, npart = _tc_mid(eps, zp0, zn0, yp0, yn0, dp, dn, x0, egn)

    zp1, zn1 = _prop_l1(y1p, edges_p, y1n, edges_p, zeros64)

    po, no = _tc_fin(eps, zp1, zn1, y1p, y1n, dp, pp, npart)

    pos_emb = jnp.concatenate([po[:HALF], po[HALFP:HALFP + HALF]], axis=0)
    neg_emb = jnp.concatenate([no[:HALF], no[HALFP:HALFP + HALF]], axis=0)
    return pos_emb, neg_emb
